# Initial kernel scaffold; baseline (speedup 1.0000x reference)
import jax
import jax.numpy as jnp
from jax.experimental import pallas as pl


def kernel(x, edge_index, edge_type, gene_emb, W1, root1, b1, W2, root2, b2, lw1, lb1, lw2, lb2):
    raise NotImplementedError("write your pallas kernel here")


# speedup vs baseline: 1.3834x; 1.3834x over previous
import dataclasses

import jax
import jax.numpy as jnp
from jax import lax
from jax.experimental import pallas as pl
from jax.experimental.pallas import tpu as pltpu
from jax.experimental.pallas import tpu_sc as plsc

N = 10000
D_IN = 1613
DIM1 = 1340
DIM2 = 920
DIM3 = 740
E = 48000
R = 4

FW = 128
O1 = 1408
F1 = O1 // FW
O2 = 1024
F2 = O2 // FW

NSUB = 16
BATCH = 128
NB = 24
SUB_E = NB * BATCH
EP = NSUB * SUB_E
ROWS = R * N
TRASH = ROWS
NCH = 6
CH = 6912
A_ROWS = NCH * CH
SUB_C = CH // NSUB
ZROWS = 144

MM1_TILE = 400
MM2_TILE = 200
HEAD_TILE = 400


def _sc_aggregate(gflat, gsel, ssel, zrows, num_f, ones):
    half = (num_f + 1) // 2
    mesh = plsc.VectorSubcoreMesh(core_axis_name="c", subcore_axis_name="s")
    out_type = [jax.ShapeDtypeStruct((num_f, A_ROWS, FW), jnp.float32)]
    if ones is not None:
        out_type.append(jax.ShapeDtypeStruct((A_ROWS, FW), jnp.float32))

    def body(*refs):
        if ones is not None:
            (g_hbm, gsel_hbm, ssel_hbm, z_hbm, ones_hbm, a_hbm, cnt_hbm,
             gb_v, gf_v, s_v, sl_v, b0, b1, z_v, acc, sem0, sem1) = refs
        else:
            (g_hbm, gsel_hbm, ssel_hbm, z_hbm, a_hbm,
             gb_v, gf_v, s_v, sl_v, b0, b1, z_v, acc, sem0, sem1) = refs
        cid = lax.axis_index("c")
        sid = lax.axis_index("s")
        pltpu.sync_copy(gsel_hbm.at[sid], gb_v)
        pltpu.sync_copy(ssel_hbm.at[sid], s_v)
        pltpu.sync_copy(z_hbm, z_v)

        def mm_step(k, mm):
            v = s_v[k // 8, pl.ds((k % 8) * 16, 16)]
            return (jnp.minimum(mm[0], jnp.min(v)),
                    jnp.maximum(mm[1], jnp.max(v)))

        wmin, wmax = lax.fori_loop(0, NB * 8, mm_step,
                                   (jnp.int32(2 ** 30), jnp.int32(-1)))

        def zero_acc():
            @pl.loop(0, SUB_C // ZROWS)
            def _(i):
                pltpu.sync_copy(
                    z_v, acc.at[pl.ds(sid * SUB_C + i * ZROWS, ZROWS)])

        def sweep(src_of):
            pltpu.async_copy(src_of(0), b0, sem0)

            @pl.loop(0, NB, step=2)
            def _(j):
                pltpu.make_async_copy(src_of(j), b0, sem0).wait()
                pltpu.async_copy(src_of(j + 1), b1, sem1)
                pltpu.sync_copy(b0, acc.at[sl_v.at[j]], add=True)
                pltpu.make_async_copy(src_of(j + 1), b1, sem1).wait()

                @pl.when(j + 2 < NB)
                def _():
                    pltpu.async_copy(src_of(j + 2), b0, sem0)

                pltpu.sync_copy(b1, acc.at[sl_v.at[j + 1]], add=True)

        @pl.loop(0, NCH)
        def _(ch):
            lo = ch * CH
            @pl.loop(0, NB)
            def _(j):
                @pl.loop(0, BATCH, step=16)
                def _(k):
                    sv = s_v[j, pl.ds(k, 16)]
                    loc = sv - lo
                    ok = (loc >= 0) & (loc < CH)
                    sl_v[j, pl.ds(k, 16)] = jnp.where(ok, loc, CH)

            hit = (wmax >= lo) & (wmin < lo + CH)

            @pl.loop(0, half)
            def _(jf):
                f = cid * half + jf

                @pl.when(f < num_f)
                def _():
                    @pl.loop(0, SUB_E, step=16)
                    def _(i):
                        gf_v[pl.ds(i, 16)] = gb_v[pl.ds(i, 16)] + f * (5 * N)

                    zero_acc()
                    plsc.subcore_barrier()

                    @pl.when(hit)
                    def _():
                        sweep(lambda j: g_hbm.at[
                            gf_v.at[pl.ds(j * BATCH, BATCH)]])

                    plsc.subcore_barrier()
                    pltpu.sync_copy(
                        acc.at[pl.ds(sid * SUB_C, SUB_C)],
                        a_hbm.at[f, pl.ds(lo + sid * SUB_C, SUB_C)])

            if ones is not None:
                @pl.when(cid == 1)
                def _():
                    zero_acc()
                    pltpu.sync_copy(ones_hbm, b0)
                    plsc.subcore_barrier()

                    @pl.when(hit)
                    def _():
                        @pl.loop(0, NB)
                        def _(j):
                            pltpu.sync_copy(b0, acc.at[sl_v.at[j]],
                                            add=True)

                    plsc.subcore_barrier()
                    pltpu.sync_copy(
                        acc.at[pl.ds(sid * SUB_C, SUB_C)],
                        cnt_hbm.at[pl.ds(lo + sid * SUB_C, SUB_C)])

    scratch = [
        pltpu.VMEM((SUB_E,), jnp.int32),
        pltpu.VMEM((SUB_E,), jnp.int32),
        pltpu.VMEM((NB, BATCH), jnp.int32),
        pltpu.VMEM((NB, BATCH), jnp.int32),
        pltpu.VMEM((BATCH, FW), jnp.float32),
        pltpu.VMEM((BATCH, FW), jnp.float32),
        pltpu.VMEM((ZROWS, FW), jnp.float32),
        pltpu.VMEM_SHARED((CH + 16, FW), jnp.float32),
        pltpu.SemaphoreType.DMA,
        pltpu.SemaphoreType.DMA,
    ]
    cp = pltpu.CompilerParams()
    if "needs_layout_passes" in pltpu.CompilerParams.__dataclass_fields__:
        cp = dataclasses.replace(cp, needs_layout_passes=False)
    kern = pl.kernel(body, out_type=out_type, mesh=mesh,
                     scratch_types=scratch, compiler_params=cp)
    if ones is not None:
        return kern(gflat, gsel, ssel, zrows, ones)
    return kern(gflat, gsel, ssel, zrows)


def _dot(a, b):
    return lax.dot_general(a, b, (((1,), (0,)), ((), ())),
                           preferred_element_type=jnp.float32,
                           precision=lax.Precision.HIGHEST)


def _mm1(h, wcat):
    def body(h_ref, w_ref, o_ref):
        res = _dot(h_ref[...], w_ref[0])
        for f in range(F1):
            o_ref[f, 0] = res[:, f * FW:(f + 1) * FW]

    return pl.pallas_call(
        body,
        grid=(5, N // MM1_TILE),
        in_specs=[
            pl.BlockSpec((MM1_TILE, D_IN), lambda r, i: (i, 0)),
            pl.BlockSpec((1, D_IN, O1), lambda r, i: (r, 0, 0)),
        ],
        out_specs=pl.BlockSpec((F1, 1, MM1_TILE, FW),
                               lambda r, i: (0, r, i, 0)),
        out_shape=jax.ShapeDtypeStruct((F1, 5, N, FW), jnp.float32),
    )(h, wcat)


def _a_specs(num_f, tile):
    def mk(r):
        return pl.BlockSpec((num_f, tile, FW),
                            lambda i, r=r: (0, (r * N) // tile + i, 0))
    return [mk(r) for r in range(R)]


def _finalize(a_refs, root_ref, invc_ref, b_ref, num_f):
    cols = []
    for f in range(num_f):
        acc = root_ref[f, 0] + b_ref[0, f * FW:(f + 1) * FW][None, :]
        for r in range(R):
            acc = acc + a_refs[r][f] * invc_ref[:, r:r + 1]
        cols.append(jnp.maximum(acc, 0.0))
    return jnp.concatenate(cols, axis=1)


def _mm2(A1, G1, invcp, b1p, w2d):
    def body(a0, a1, a2, a3, g1_ref, invc_ref, b_ref, w_ref, o_ref):
        h2 = _finalize((a0, a1, a2, a3), g1_ref, invc_ref, b_ref, F1)
        res = _dot(h2, w_ref[...])
        for r in range(5):
            for f in range(F2):
                o_ref[f, r] = res[:, r * O2 + f * FW:r * O2 + (f + 1) * FW]

    return pl.pallas_call(
        body,
        grid=(N // MM2_TILE,),
        in_specs=_a_specs(F1, MM2_TILE) + [
            pl.BlockSpec((F1, 1, MM2_TILE, FW), lambda i: (0, 4, i, 0)),
            pl.BlockSpec((MM2_TILE, 8), lambda i: (i, 0)),
            pl.BlockSpec((1, O1), lambda i: (0, 0)),
            pl.BlockSpec((O1, 5 * O2), lambda i: (0, 0)),
        ],
        out_specs=pl.BlockSpec((F2, 5, MM2_TILE, FW),
                               lambda i: (0, 0, i, 0)),
        out_shape=jax.ShapeDtypeStruct((F2, 5, N, FW), jnp.float32),
    )(A1, A1, A1, A1, G1, invcp, b1p, w2d)


def _head(A2, G2, invcp, b2p, lw1p, lb1, lw2p, lb2p):
    def body(a0, a1, a2, a3, g2_ref, invc_ref, b_ref, w1_ref, c1_ref,
             w2_ref, c2_ref, out_ref, emb_ref):
        h3 = _finalize((a0, a1, a2, a3), g2_ref, invc_ref, b_ref, F2)
        emb = jnp.maximum(_dot(h3, w1_ref[...]) + c1_ref[0][None, :], 0.0)
        emb_ref[...] = emb
        lg = _dot(emb, w2_ref[...]) + c2_ref[0][None, :]
        x0 = lg[:, 0:1]
        x1 = lg[:, 1:2]
        m = jnp.maximum(x0, x1)
        lse = m + jnp.log(jnp.exp(x0 - m) + jnp.exp(x1 - m))
        out_ref[...] = jnp.concatenate([x0 - lse, x1 - lse], axis=1)

    return pl.pallas_call(
        body,
        grid=(N // HEAD_TILE,),
        in_specs=_a_specs(F2, HEAD_TILE) + [
            pl.BlockSpec((F2, 1, HEAD_TILE, FW), lambda i: (0, 4, i, 0)),
            pl.BlockSpec((HEAD_TILE, 8), lambda i: (i, 0)),
            pl.BlockSpec((1, O2), lambda i: (0, 0)),
            pl.BlockSpec((O2, DIM3), lambda i: (0, 0)),
            pl.BlockSpec((1, DIM3), lambda i: (0, 0)),
            pl.BlockSpec((DIM3, 128), lambda i: (0, 0)),
            pl.BlockSpec((1, 128), lambda i: (0, 0)),
        ],
        out_specs=[
            pl.BlockSpec((HEAD_TILE, 2), lambda i: (i, 0)),
            pl.BlockSpec((HEAD_TILE, DIM3), lambda i: (i, 0)),
        ],
        out_shape=[
            jax.ShapeDtypeStruct((N, 2), jnp.float32),
            jax.ShapeDtypeStruct((N, DIM3), jnp.float32),
        ],
    )(A2, A2, A2, A2, G2, invcp, b2p, lw1p, lb1, lw2p, lb2p)


def kernel(x, edge_index, edge_type, gene_emb, W1, root1, b1, W2, root2,
           b2, lw1, lb1, lw2, lb2):
    f32 = jnp.float32
    h = jnp.concatenate([x, gene_emb], axis=0)
    src = edge_index[0]
    dst = edge_index[1]
    t = edge_type
    seg = t * N + dst
    perm = jnp.argsort(seg)
    seg_s = seg[perm]
    base_s = (t * N + src)[perm]
    pad = EP - E
    seg_p = jnp.concatenate([seg_s, jnp.full((pad,), TRASH, jnp.int32)])
    base_p = jnp.concatenate([base_s, jnp.zeros((pad,), jnp.int32)])
    ssel = seg_p.reshape(NSUB, NB, BATCH)
    gsel = base_p.reshape(NSUB, SUB_E)
    zrows = jnp.zeros((ZROWS, FW), f32)
    ones = jnp.ones((BATCH, FW), f32)

    wcat1 = jnp.pad(jnp.concatenate([W1, root1[None]], axis=0),
                    ((0, 0), (0, 0), (0, O1 - DIM1)))
    b1p = jnp.pad(b1, (0, O1 - DIM1)).reshape(1, O1)
    wcat2 = jnp.pad(jnp.concatenate([W2, root2[None]], axis=0),
                    ((0, 0), (0, O1 - DIM1), (0, O2 - DIM2)))
    w2d = wcat2.transpose(1, 0, 2).reshape(O1, 5 * O2)
    b2p = jnp.pad(b2, (0, O2 - DIM2)).reshape(1, O2)
    lw1p = jnp.pad(lw1, ((0, O2 - DIM2), (0, 0)))
    lb1r = lb1.reshape(1, DIM3)
    lw2p = jnp.pad(lw2, ((0, 0), (0, 126)))
    lb2p = jnp.pad(lb2, (0, 126)).reshape(1, 128)

    G1 = _mm1(h, wcat1)
    A1, cnt = _sc_aggregate(G1.reshape(F1 * 5 * N, FW), gsel, ssel,
                            zrows, F1, ones)
    invc = 1.0 / jnp.maximum(cnt[:ROWS, 0], 1.0)
    invcp = jnp.pad(invc.reshape(R, N).T, ((0, 0), (0, 4)))
    G2 = _mm2(A1, G1, invcp, b1p, w2d)
    (A2,) = _sc_aggregate(G2.reshape(F2 * 5 * N, FW), gsel, ssel,
                          zrows, F2, None)
    return _head(A2, G2, invcp, b2p, lw1p, lb1r, lw2p, lb2p)


# speedup vs baseline: 2.1818x; 1.5771x over previous
import dataclasses

import jax
import jax.numpy as jnp
from jax import lax
from jax.experimental import pallas as pl
from jax.experimental.pallas import tpu as pltpu
from jax.experimental.pallas import tpu_sc as plsc

N = 10000
D_IN = 1613
DIM1 = 1340
DIM2 = 920
DIM3 = 740
E = 48000
R = 4

FW = 128
O1 = 1408
F1 = O1 // FW
O2 = 1024
F2 = O2 // FW

NSUB = 16
BATCH = 128
NB = 24
SUB_E = NB * BATCH
EP = NSUB * SUB_E
ROWS = R * N
TRASH = ROWS
NCH = 6
CH = 6912
A_ROWS = NCH * CH
SUB_C = CH // NSUB
ZROWS = 144

MM1_TILE = 400
MM2_TILE = 200
HEAD_TILE = 400


def _sc_aggregate(gflat, gsel, ssel, zrows, num_f, ones):
    half = (num_f + 1) // 2
    mesh = plsc.VectorSubcoreMesh(core_axis_name="c", subcore_axis_name="s")
    out_type = [jax.ShapeDtypeStruct((num_f, A_ROWS, FW), jnp.float32)]
    if ones is not None:
        out_type.append(jax.ShapeDtypeStruct((A_ROWS, FW), jnp.float32))

    def body(*refs):
        if ones is not None:
            (g_hbm, gsel_hbm, ssel_hbm, z_hbm, ones_hbm, a_hbm, cnt_hbm,
             gb_v, gf_v, s_v, sl_v, b0, b1, z_v, acc,
             gsem0, gsem1, ssem0, ssem1) = refs
        else:
            (g_hbm, gsel_hbm, ssel_hbm, z_hbm, a_hbm,
             gb_v, gf_v, s_v, sl_v, b0, b1, z_v, acc,
             gsem0, gsem1, ssem0, ssem1) = refs
        cid = lax.axis_index("c")
        sid = lax.axis_index("s")
        pltpu.sync_copy(gsel_hbm.at[sid], gb_v)
        pltpu.sync_copy(ssel_hbm.at[sid], s_v)
        pltpu.sync_copy(z_hbm, z_v)

        def mm_step(k, mm):
            v = s_v[k // 8, pl.ds((k % 8) * 16, 16)]
            return (jnp.minimum(mm[0], jnp.min(v)),
                    jnp.maximum(mm[1], jnp.max(v)))

        wmin, wmax = lax.fori_loop(0, NB * 8, mm_step,
                                   (jnp.int32(2 ** 30), jnp.int32(-1)))

        def zero_acc():
            @pl.loop(0, SUB_C // ZROWS)
            def _(i):
                pltpu.sync_copy(
                    z_v, acc.at[pl.ds(sid * SUB_C + i * ZROWS, ZROWS)])

        def sweep(src_of):
            pltpu.async_copy(src_of(0), b0, gsem0)

            @pl.loop(0, NB, step=2)
            def _(j):
                pltpu.make_async_copy(src_of(j), b0, gsem0).wait()
                pltpu.async_copy(src_of(j + 1), b1, gsem1)
                pltpu.sync_copy(b0, acc.at[sl_v.at[j]], add=True)
                pltpu.make_async_copy(src_of(j + 1), b1, gsem1).wait()

                @pl.when(j + 2 < NB)
                def _():
                    pltpu.async_copy(src_of(j + 2), b0, gsem0)

                pltpu.sync_copy(b1, acc.at[sl_v.at[j + 1]], add=True)

        @pl.loop(0, NCH)
        def _(ch):
            lo = ch * CH
            @pl.loop(0, NB)
            def _(j):
                @pl.loop(0, BATCH, step=16)
                def _(k):
                    sv = s_v[j, pl.ds(k, 16)]
                    loc = sv - lo
                    ok = (loc >= 0) & (loc < CH)
                    sl_v[j, pl.ds(k, 16)] = jnp.where(ok, loc, CH)

            hit = (wmax >= lo) & (wmin < lo + CH)

            @pl.loop(0, half)
            def _(jf):
                f = cid * half + jf

                @pl.when(f < num_f)
                def _():
                    @pl.loop(0, SUB_E, step=16)
                    def _(i):
                        gf_v[pl.ds(i, 16)] = gb_v[pl.ds(i, 16)] + f * (5 * N)

                    zero_acc()
                    plsc.subcore_barrier()

                    @pl.when(hit)
                    def _():
                        sweep(lambda j: g_hbm.at[
                            gf_v.at[pl.ds(j * BATCH, BATCH)]])

                    plsc.subcore_barrier()
                    pltpu.sync_copy(
                        acc.at[pl.ds(sid * SUB_C, SUB_C)],
                        a_hbm.at[f, pl.ds(lo + sid * SUB_C, SUB_C)])

            if ones is not None:
                @pl.when(cid == 1)
                def _():
                    zero_acc()
                    pltpu.sync_copy(ones_hbm, b0)
                    plsc.subcore_barrier()

                    @pl.when(hit)
                    def _():
                        @pl.loop(0, NB)
                        def _(j):
                            pltpu.sync_copy(b0, acc.at[sl_v.at[j]],
                                            add=True)

                    plsc.subcore_barrier()
                    pltpu.sync_copy(
                        acc.at[pl.ds(sid * SUB_C, SUB_C)],
                        cnt_hbm.at[pl.ds(lo + sid * SUB_C, SUB_C)])

    scratch = [
        pltpu.VMEM((SUB_E,), jnp.int32),
        pltpu.VMEM((SUB_E,), jnp.int32),
        pltpu.VMEM((NB, BATCH), jnp.int32),
        pltpu.VMEM((NB, BATCH), jnp.int32),
        pltpu.VMEM((BATCH, FW), jnp.float32),
        pltpu.VMEM((BATCH, FW), jnp.float32),
        pltpu.VMEM((ZROWS, FW), jnp.float32),
        pltpu.VMEM_SHARED((CH + 16, FW), jnp.float32),
        pltpu.SemaphoreType.DMA,
        pltpu.SemaphoreType.DMA,
        pltpu.SemaphoreType.DMA,
        pltpu.SemaphoreType.DMA,
    ]
    cp = pltpu.CompilerParams()
    if "needs_layout_passes" in pltpu.CompilerParams.__dataclass_fields__:
        cp = dataclasses.replace(cp, needs_layout_passes=False)
    kern = pl.kernel(body, out_type=out_type, mesh=mesh,
                     scratch_types=scratch, compiler_params=cp)
    if ones is not None:
        return kern(gflat, gsel, ssel, zrows, ones)
    return kern(gflat, gsel, ssel, zrows)


def _dot(a, b):
    return lax.dot_general(a, b, (((1,), (0,)), ((), ())),
                           preferred_element_type=jnp.float32,
                           precision=lax.Precision.DEFAULT)


def _mm1(h, wcat):
    def body(h_ref, w_ref, o_ref):
        res = _dot(h_ref[...], w_ref[0])
        for f in range(F1):
            o_ref[f, 0] = res[:, f * FW:(f + 1) * FW]

    return pl.pallas_call(
        body,
        grid=(5, N // MM1_TILE),
        in_specs=[
            pl.BlockSpec((MM1_TILE, D_IN), lambda r, i: (i, 0)),
            pl.BlockSpec((1, D_IN, O1), lambda r, i: (r, 0, 0)),
        ],
        out_specs=pl.BlockSpec((F1, 1, MM1_TILE, FW),
                               lambda r, i: (0, r, i, 0)),
        out_shape=jax.ShapeDtypeStruct((F1, 5, N, FW), jnp.float32),
    )(h, wcat)


def _a_specs(num_f, tile):
    def mk(r):
        return pl.BlockSpec((num_f, tile, FW),
                            lambda i, r=r: (0, (r * N) // tile + i, 0))
    return [mk(r) for r in range(R)]


def _finalize(a_refs, root_ref, invc_ref, b_ref, num_f):
    cols = []
    for f in range(num_f):
        acc = root_ref[f, 0] + b_ref[0, f * FW:(f + 1) * FW][None, :]
        for r in range(R):
            acc = acc + a_refs[r][f] * invc_ref[:, r:r + 1]
        cols.append(jnp.maximum(acc, 0.0))
    return jnp.concatenate(cols, axis=1)


def _mm2(A1, G1, invcp, b1p, w2d):
    def body(a0, a1, a2, a3, g1_ref, invc_ref, b_ref, w_ref, o_ref):
        h2 = _finalize((a0, a1, a2, a3), g1_ref, invc_ref, b_ref, F1)
        res = _dot(h2, w_ref[...])
        for r in range(5):
            for f in range(F2):
                o_ref[f, r] = res[:, r * O2 + f * FW:r * O2 + (f + 1) * FW]

    return pl.pallas_call(
        body,
        grid=(N // MM2_TILE,),
        in_specs=_a_specs(F1, MM2_TILE) + [
            pl.BlockSpec((F1, 1, MM2_TILE, FW), lambda i: (0, 4, i, 0)),
            pl.BlockSpec((MM2_TILE, 8), lambda i: (i, 0)),
            pl.BlockSpec((1, O1), lambda i: (0, 0)),
            pl.BlockSpec((O1, 5 * O2), lambda i: (0, 0)),
        ],
        out_specs=pl.BlockSpec((F2, 5, MM2_TILE, FW),
                               lambda i: (0, 0, i, 0)),
        out_shape=jax.ShapeDtypeStruct((F2, 5, N, FW), jnp.float32),
    )(A1, A1, A1, A1, G1, invcp, b1p, w2d)


def _head(A2, G2, invcp, b2p, lw1p, lb1, lw2p, lb2p):
    def body(a0, a1, a2, a3, g2_ref, invc_ref, b_ref, w1_ref, c1_ref,
             w2_ref, c2_ref, out_ref, emb_ref):
        h3 = _finalize((a0, a1, a2, a3), g2_ref, invc_ref, b_ref, F2)
        emb = jnp.maximum(_dot(h3, w1_ref[...]) + c1_ref[0][None, :], 0.0)
        emb_ref[...] = emb
        lg = _dot(emb, w2_ref[...]) + c2_ref[0][None, :]
        x0 = lg[:, 0:1]
        x1 = lg[:, 1:2]
        m = jnp.maximum(x0, x1)
        lse = m + jnp.log(jnp.exp(x0 - m) + jnp.exp(x1 - m))
        out_ref[...] = jnp.concatenate([x0 - lse, x1 - lse], axis=1)

    return pl.pallas_call(
        body,
        grid=(N // HEAD_TILE,),
        in_specs=_a_specs(F2, HEAD_TILE) + [
            pl.BlockSpec((F2, 1, HEAD_TILE, FW), lambda i: (0, 4, i, 0)),
            pl.BlockSpec((HEAD_TILE, 8), lambda i: (i, 0)),
            pl.BlockSpec((1, O2), lambda i: (0, 0)),
            pl.BlockSpec((O2, DIM3), lambda i: (0, 0)),
            pl.BlockSpec((1, DIM3), lambda i: (0, 0)),
            pl.BlockSpec((DIM3, 128), lambda i: (0, 0)),
            pl.BlockSpec((1, 128), lambda i: (0, 0)),
        ],
        out_specs=[
            pl.BlockSpec((HEAD_TILE, 2), lambda i: (i, 0)),
            pl.BlockSpec((HEAD_TILE, DIM3), lambda i: (i, 0)),
        ],
        out_shape=[
            jax.ShapeDtypeStruct((N, 2), jnp.float32),
            jax.ShapeDtypeStruct((N, DIM3), jnp.float32),
        ],
    )(A2, A2, A2, A2, G2, invcp, b2p, lw1p, lb1, lw2p, lb2p)


def kernel(x, edge_index, edge_type, gene_emb, W1, root1, b1, W2, root2,
           b2, lw1, lb1, lw2, lb2):
    f32 = jnp.float32
    h = jnp.concatenate([x, gene_emb], axis=0)
    src = edge_index[0]
    dst = edge_index[1]
    t = edge_type
    seg = t * N + dst
    seg_s, base_s = lax.sort((seg, t * N + src), num_keys=1)
    pad = EP - E
    seg_p = jnp.concatenate([seg_s, jnp.full((pad,), TRASH, jnp.int32)])
    base_p = jnp.concatenate([base_s, jnp.zeros((pad,), jnp.int32)])
    ssel = seg_p.reshape(NSUB, NB, BATCH)
    gsel = base_p.reshape(NSUB, SUB_E)
    zrows = jnp.zeros((ZROWS, FW), f32)
    ones = jnp.ones((BATCH, FW), f32)

    wcat1 = jnp.pad(jnp.concatenate([W1, root1[None]], axis=0),
                    ((0, 0), (0, 0), (0, O1 - DIM1)))
    b1p = jnp.pad(b1, (0, O1 - DIM1)).reshape(1, O1)
    wcat2 = jnp.pad(jnp.concatenate([W2, root2[None]], axis=0),
                    ((0, 0), (0, O1 - DIM1), (0, O2 - DIM2)))
    w2d = wcat2.transpose(1, 0, 2).reshape(O1, 5 * O2)
    b2p = jnp.pad(b2, (0, O2 - DIM2)).reshape(1, O2)
    lw1p = jnp.pad(lw1, ((0, O2 - DIM2), (0, 0)))
    lb1r = lb1.reshape(1, DIM3)
    lw2p = jnp.pad(lw2, ((0, 0), (0, 126)))
    lb2p = jnp.pad(lb2, (0, 126)).reshape(1, 128)

    G1 = _mm1(h, wcat1)
    A1, cnt = _sc_aggregate(G1.reshape(F1 * 5 * N, FW), gsel, ssel,
                            zrows, F1, ones)
    invc = 1.0 / jnp.maximum(cnt[:ROWS, 0], 1.0)
    invcp = jnp.pad(invc.reshape(R, N).T, ((0, 0), (0, 4)))
    G2 = _mm2(A1, G1, invcp, b1p, w2d)
    (A2,) = _sc_aggregate(G2.reshape(F2 * 5 * N, FW), gsel, ssel,
                          zrows, F2, None)
    return _head(A2, G2, invcp, b2p, lw1p, lb1r, lw2p, lb2p)
